# Initial kernel scaffold; baseline (speedup 1.0000x reference)
import jax
import jax.numpy as jnp
from jax.experimental import pallas as pl


def kernel(x, edge_index, W, b):
    raise NotImplementedError("write your pallas kernel here")


# speedup vs baseline: 5.4499x; 5.4499x over previous
import functools

import jax
import jax.numpy as jnp
from jax import lax
from jax.experimental import pallas as pl
from jax.experimental.pallas import tpu as pltpu
from jax.experimental.pallas import tpu_sc as plsc

N_NODES = 10000
N_EDGES = 320000
D = 128

_NC = 2
_NS = 16
_NW = _NC * _NS
_EPW = N_EDGES // _NW
_K = 80
_CHUNKS = _EPW // _K
_RCHUNKS = N_NODES // _K


def _sc_body(x_hbm, src_hbm, dst_hbm, out_hbm, idx_src, idx_dst, rows_v, acc, sem):
    c = lax.axis_index("c")
    s = lax.axis_index("s")
    wid = c * _NS + s

    zero = jnp.zeros((16,), jnp.float32)

    def _zrow(i, carry):
        for j in range(D // 16):
            rows_v[i, pl.ds(j * 16, 16)] = zero
        return carry

    lax.fori_loop(0, _K, _zrow, 0)
    for k in range((_RCHUNKS + _NS - 1) // _NS):
        cid = k * _NS + s

        @pl.when(cid < _RCHUNKS)
        def _():
            off = pl.multiple_of(cid * _K, 8)
            pltpu.sync_copy(rows_v, acc.at[pl.ds(off, _K)])

    plsc.subcore_barrier()

    ebase = wid * _EPW

    def _chunk(i, carry):
        off = ebase + i * _K
        pltpu.sync_copy(src_hbm.at[pl.ds(off, _K)], idx_src)
        pltpu.sync_copy(dst_hbm.at[pl.ds(off, _K)], idx_dst)
        pltpu.async_copy(x_hbm.at[idx_src], rows_v, sem).wait()
        pltpu.sync_copy(rows_v, acc.at[idx_dst], add=True)
        return carry

    lax.fori_loop(0, _CHUNKS, _chunk, 0)
    plsc.subcore_barrier()

    for k in range((_RCHUNKS + _NS - 1) // _NS):
        cid = k * _NS + s

        @pl.when(cid < _RCHUNKS)
        def _():
            off = pl.multiple_of(cid * _K, 8)
            pltpu.sync_copy(acc.at[pl.ds(off, _K)], out_hbm.at[c, pl.ds(off, _K)])


def _sc_aggregate(x, src, dst):
    mesh = plsc.VectorSubcoreMesh(core_axis_name="c", subcore_axis_name="s")
    return pl.kernel(
        _sc_body,
        out_type=jax.ShapeDtypeStruct((_NC, N_NODES, D), jnp.float32),
        mesh=mesh,
        scratch_types=[
            pltpu.VMEM((_K,), jnp.int32),
            pltpu.VMEM((_K,), jnp.int32),
            pltpu.VMEM((_K, D), jnp.float32),
            pltpu.VMEM_SHARED((N_NODES, D), jnp.float32),
            pltpu.SemaphoreType.DMA,
        ],
    )(x, src, dst)


def _tc_body(x_ref, pa_ref, pb_ref, w1_ref, w2_ref, b_ref, o_ref):
    ng = pa_ref[...] + pb_ref[...]
    o_ref[...] = (
        jnp.dot(x_ref[...], w1_ref[...], preferred_element_type=jnp.float32)
        + jnp.dot(ng, w2_ref[...], preferred_element_type=jnp.float32)
        + b_ref[...]
    )


def _tc_linear(x, pa, pb, w1t, w2t, b2d):
    blk = 1000
    grid = (N_NODES // blk,)
    return pl.pallas_call(
        _tc_body,
        grid=grid,
        in_specs=[
            pl.BlockSpec((blk, D), lambda i: (i, 0)),
            pl.BlockSpec((blk, D), lambda i: (i, 0)),
            pl.BlockSpec((blk, D), lambda i: (i, 0)),
            pl.BlockSpec((D, D), lambda i: (0, 0)),
            pl.BlockSpec((D, D), lambda i: (0, 0)),
            pl.BlockSpec((1, D), lambda i: (0, 0)),
        ],
        out_specs=pl.BlockSpec((blk, D), lambda i: (i, 0)),
        out_shape=jax.ShapeDtypeStruct((N_NODES, D), jnp.float32),
    )(x, pa, pb, w1t, w2t, b2d)


@jax.jit
def kernel(x, edge_index, W, b):
    src = edge_index[0].astype(jnp.int32)
    dst = edge_index[1].astype(jnp.int32)
    partials = _sc_aggregate(x, src, dst)
    w1t = W[:, :D].T
    w2t = W[:, D:].T
    return _tc_linear(x, partials[0], partials[1], w1t, w2t, b.reshape(1, D))


# speedup vs baseline: 8.6112x; 1.5801x over previous
import jax
import jax.numpy as jnp
from jax import lax
from jax.experimental import pallas as pl
from jax.experimental.pallas import tpu as pltpu
from jax.experimental.pallas import tpu_sc as plsc

N_NODES = 10000
N_EDGES = 320000
D = 128
DH = D // 2

_NC = 2
_NS = 16
_EPW = N_EDGES // _NS
_K = 80
_CHUNKS = _EPW // _K
_NB = 10
_RCHUNKS = N_NODES // _K


def _sc_body(x_hbm, src_hbm, dst_hbm, out_hbm, *refs):
    isrc = refs[:_NB]
    idst = refs[_NB:2 * _NB]
    rows = refs[2 * _NB:3 * _NB]
    acc, sem_i, sem_g, sem_s = refs[3 * _NB:]
    c = lax.axis_index("c")
    s = lax.axis_index("s")
    rows_v = rows[0]

    zero = jnp.zeros((16,), jnp.float32)

    def _zrow(i, carry):
        for j in range(DH // 16):
            rows_v[i, pl.ds(j * 16, 16)] = zero
        return carry

    lax.fori_loop(0, _K, _zrow, 0)
    for k in range((_RCHUNKS + _NS - 1) // _NS):
        cid = k * _NS + s

        @pl.when(cid < _RCHUNKS)
        def _():
            off = pl.multiple_of(cid * _K, 8)
            pltpu.sync_copy(rows_v, acc.at[pl.ds(off, _K)])

    plsc.subcore_barrier()

    xh = x_hbm.at[c]
    ebase = s * _EPW

    def _round(g, carry):
        base = ebase + g * (_NB * _K)
        ids = []
        for b in range(_NB):
            off = base + b * _K
            ids.append(pltpu.async_copy(src_hbm.at[pl.ds(off, _K)], isrc[b], sem_i))
            ids.append(pltpu.async_copy(dst_hbm.at[pl.ds(off, _K)], idst[b], sem_i))
        for d in ids:
            d.wait()
        gds = [pltpu.async_copy(xh.at[isrc[b]], rows[b], sem_g)
               for b in range(_NB)]
        for d in gds:
            d.wait()
        sds = [pltpu.async_copy(rows[b], acc.at[idst[b]], sem_s, add=True)
               for b in range(_NB)]
        for d in sds:
            d.wait()
        return carry

    lax.fori_loop(0, _CHUNKS // _NB, _round, 0)
    plsc.subcore_barrier()

    for k in range((_RCHUNKS + _NS - 1) // _NS):
        cid = k * _NS + s

        @pl.when(cid < _RCHUNKS)
        def _():
            off = pl.multiple_of(cid * _K, 8)
            pltpu.sync_copy(acc.at[pl.ds(off, _K)], out_hbm.at[c, pl.ds(off, _K)])


def _sc_aggregate(xsplit, src, dst):
    mesh = plsc.VectorSubcoreMesh(core_axis_name="c", subcore_axis_name="s")
    return pl.kernel(
        _sc_body,
        out_type=jax.ShapeDtypeStruct((_NC, N_NODES, DH), jnp.float32),
        mesh=mesh,
        scratch_types=(
            [pltpu.VMEM((_K,), jnp.int32) for _ in range(2 * _NB)]
            + [pltpu.VMEM((_K, DH), jnp.float32) for _ in range(_NB)]
            + [pltpu.VMEM_SHARED((N_NODES, DH), jnp.float32),
               pltpu.SemaphoreType.DMA,
               pltpu.SemaphoreType.DMA,
               pltpu.SemaphoreType.DMA]
        ),
        compiler_params=pltpu.CompilerParams(use_tc_tiling_on_sc=False),
    )(xsplit, src, dst)


def _tc_body(x_ref, pa_ref, pb_ref, w1_ref, w2_ref, b_ref, o_ref):
    ng = jnp.concatenate([pa_ref[...], pb_ref[...]], axis=1)
    o_ref[...] = (
        jnp.dot(x_ref[...], w1_ref[...], preferred_element_type=jnp.float32)
        + jnp.dot(ng, w2_ref[...], preferred_element_type=jnp.float32)
        + b_ref[...]
    )


def _tc_linear(x, pa, pb, w1t, w2t, b2d):
    blk = 1000
    grid = (N_NODES // blk,)
    return pl.pallas_call(
        _tc_body,
        grid=grid,
        in_specs=[
            pl.BlockSpec((blk, D), lambda i: (i, 0)),
            pl.BlockSpec((blk, DH), lambda i: (i, 0)),
            pl.BlockSpec((blk, DH), lambda i: (i, 0)),
            pl.BlockSpec((D, D), lambda i: (0, 0)),
            pl.BlockSpec((D, D), lambda i: (0, 0)),
            pl.BlockSpec((1, D), lambda i: (0, 0)),
        ],
        out_specs=pl.BlockSpec((blk, D), lambda i: (i, 0)),
        out_shape=jax.ShapeDtypeStruct((N_NODES, D), jnp.float32),
    )(x, pa, pb, w1t, w2t, b2d)


@jax.jit
def kernel(x, edge_index, W, b):
    src = edge_index[0].astype(jnp.int32)
    dst = edge_index[1].astype(jnp.int32)
    xsplit = jnp.stack([x[:, :DH], x[:, DH:]])
    ng_halves = _sc_aggregate(xsplit, src, dst)
    w1t = W[:, :D].T
    w2t = W[:, D:].T
    return _tc_linear(x, ng_halves[0], ng_halves[1], w1t, w2t, b.reshape(1, D))


# speedup vs baseline: 10.8739x; 1.2628x over previous
import jax
import jax.numpy as jnp
from jax import lax
from jax.experimental import pallas as pl
from jax.experimental.pallas import tpu as pltpu
from jax.experimental.pallas import tpu_sc as plsc

N_NODES = 10000
N_EDGES = 320000
D = 128
DH = D // 2

_NC = 2
_NS = 16
_EPW = N_EDGES // _NS
_K = 80
_CHUNKS = _EPW // _K
_NB = 5
_NR = 2 * _NB
_PAIRS = _CHUNKS // _NR - 1
_RCHUNKS = N_NODES // _K


def _sc_body(x_hbm, src_hbm, dst_hbm, out_hbm, *refs):
    isrc_all = refs[0]
    idst = refs[1:1 + _NR]
    rows = refs[1 + _NR:1 + 2 * _NR]
    acc = refs[1 + 2 * _NR]
    sem_id = refs[2 + 2 * _NR:2 + 3 * _NR]
    sem_g = refs[2 + 3 * _NR:2 + 4 * _NR]
    sem_s = refs[2 + 4 * _NR:2 + 5 * _NR]
    c = lax.axis_index("c")
    s = lax.axis_index("s")
    rows_v = rows[0]

    zero = jnp.zeros((16,), jnp.float32)

    def _zrow(i, carry):
        for j in range(DH // 16):
            rows_v[i, pl.ds(j * 16, 16)] = zero
        return carry

    lax.fori_loop(0, _K, _zrow, 0)
    for k in range((_RCHUNKS + _NS - 1) // _NS):
        cid = k * _NS + s

        @pl.when(cid < _RCHUNKS)
        def _():
            off = pl.multiple_of(cid * _K, 8)
            pltpu.sync_copy(rows_v, acc.at[pl.ds(off, _K)])

    plsc.subcore_barrier()

    xh = x_hbm.at[c]
    cbase = s * _CHUNKS

    pltpu.sync_copy(src_hbm.at[pl.ds(cbase, _CHUNKS)], isrc_all)

    def _fire(slot, chunk):
        pltpu.async_copy(dst_hbm.at[cbase + chunk], idst[slot], sem_id[slot])
        pltpu.async_copy(xh.at[isrc_all.at[chunk]], rows[slot], sem_g[slot])

    def _wait_and_scatter(slot, chunk):
        pltpu.make_async_copy(xh.at[isrc_all.at[chunk]], rows[slot],
                              sem_g[slot]).wait()
        pltpu.make_async_copy(dst_hbm.at[cbase + chunk], idst[slot],
                              sem_id[slot]).wait()
        pltpu.async_copy(rows[slot], acc.at[idst[slot]], sem_s[slot], add=True)

    def _wait_scatter(slot, chunk):
        pltpu.make_async_copy(rows[slot], acc.at[idst[slot]],
                              sem_s[slot]).wait()

    for b in range(_NR):
        _fire(b, b)

    def _pair(t, carry):
        base = t * _NR
        nxt = base + _NR
        for b in range(_NB):
            _wait_and_scatter(b, base + b)
        for b in range(_NB, _NR):
            _wait_and_scatter(b, base + b)
        for b in range(_NB):
            _wait_scatter(b, base + b)
            _fire(b, nxt + b)
        for b in range(_NB, _NR):
            _wait_scatter(b, base + b)
            _fire(b, nxt + b)
        return carry

    lax.fori_loop(0, _PAIRS, _pair, 0)

    last = _PAIRS * _NR
    for b in range(_NR):
        _wait_and_scatter(b, last + b)
    for b in range(_NR):
        _wait_scatter(b, last + b)
    plsc.subcore_barrier()

    for k in range((_RCHUNKS + _NS - 1) // _NS):
        cid = k * _NS + s

        @pl.when(cid < _RCHUNKS)
        def _():
            off = pl.multiple_of(cid * _K, 8)
            pltpu.sync_copy(acc.at[pl.ds(off, _K)], out_hbm.at[c, pl.ds(off, _K)])


def _sc_aggregate(xsplit, src, dst):
    mesh = plsc.VectorSubcoreMesh(core_axis_name="c", subcore_axis_name="s")
    return pl.kernel(
        _sc_body,
        out_type=jax.ShapeDtypeStruct((_NC, N_NODES, DH), jnp.float32),
        mesh=mesh,
        scratch_types=(
            [pltpu.VMEM((_CHUNKS, _K), jnp.int32)]
            + [pltpu.VMEM((_K,), jnp.int32) for _ in range(_NR)]
            + [pltpu.VMEM((_K, DH), jnp.float32) for _ in range(_NR)]
            + [pltpu.VMEM_SHARED((N_NODES, DH), jnp.float32)]
            + [pltpu.SemaphoreType.DMA for _ in range(3 * _NR)]
        ),
        compiler_params=pltpu.CompilerParams(use_tc_tiling_on_sc=False),
    )(xsplit, src, dst)


def _tc_body(x_ref, pa_ref, pb_ref, w1_ref, w2_ref, b_ref, o_ref):
    ng = jnp.concatenate([pa_ref[...], pb_ref[...]], axis=1)
    o_ref[...] = (
        jnp.dot(x_ref[...], w1_ref[...], preferred_element_type=jnp.float32)
        + jnp.dot(ng, w2_ref[...], preferred_element_type=jnp.float32)
        + b_ref[...]
    )


def _tc_linear(x, pa, pb, w1t, w2t, b2d):
    blk = 1000
    grid = (N_NODES // blk,)
    return pl.pallas_call(
        _tc_body,
        grid=grid,
        in_specs=[
            pl.BlockSpec((blk, D), lambda i: (i, 0)),
            pl.BlockSpec((blk, DH), lambda i: (i, 0)),
            pl.BlockSpec((blk, DH), lambda i: (i, 0)),
            pl.BlockSpec((D, D), lambda i: (0, 0)),
            pl.BlockSpec((D, D), lambda i: (0, 0)),
            pl.BlockSpec((1, D), lambda i: (0, 0)),
        ],
        out_specs=pl.BlockSpec((blk, D), lambda i: (i, 0)),
        out_shape=jax.ShapeDtypeStruct((N_NODES, D), jnp.float32),
    )(x, pa, pb, w1t, w2t, b2d)


@jax.jit
def kernel(x, edge_index, W, b):
    src = edge_index[0].astype(jnp.int32).reshape(_NS * _CHUNKS, _K)
    dst = edge_index[1].astype(jnp.int32).reshape(_NS * _CHUNKS, _K)
    xsplit = jnp.stack([x[:, :DH], x[:, DH:]])
    ng_halves = _sc_aggregate(xsplit, src, dst)
    w1t = W[:, :D].T
    w2t = W[:, D:].T
    return _tc_linear(x, ng_halves[0], ng_halves[1], w1t, w2t, b.reshape(1, D))
